# baseline (speedup 1.0000x reference)
import jax
import jax.numpy as jnp
from jax import lax
from jax.experimental import pallas as pl
from jax.experimental.pallas import tpu as pltpu
from jax.experimental.pallas import tpu_sc as plsc

Nb, Na, Nnbh = 2, 5000, 16
NIN, NF, NOUT, NG = 128, 128, 128, 64

GATHER_WINDOW = 256
SC_UNITS = 32
_LN2 = 0.6931471805599453


def _ssp(v):
    t = jnp.exp(-jnp.abs(v))
    return jnp.maximum(v, 0.0) + jnp.log1p(t) - _LN2


def _sc_gather(table, idx_pad):
    n_idx = idx_pad.shape[0]
    c = table.shape[1]
    idx2 = idx_pad.reshape(1, n_idx)
    mesh = plsc.VectorSubcoreMesh(core_axis_name="c", subcore_axis_name="s")

    @pl.kernel(out_type=jax.ShapeDtypeStruct((n_idx, c), table.dtype), mesh=mesh)
    def gather_kernel(x_hbm, i_hbm, o_hbm):
        def body(i_vmem, o_vmem):
            pltpu.sync_copy(x_hbm.at[i_vmem.at[0]], o_vmem)

        pltpu.emit_pipeline(
            body,
            grid=(n_idx // GATHER_WINDOW,),
            in_specs=[pl.BlockSpec((1, GATHER_WINDOW), index_map=lambda i: (0, i))],
            out_specs=[pl.BlockSpec((GATHER_WINDOW, c), index_map=lambda i: (i, 0))],
            core_axis_name="s",
            dimension_semantics=(pltpu.PARALLEL,),
        )(i_hbm, o_hbm)

    return gather_kernel(table, idx2)


def _fused_body(xg_ref, mask_ref, fs_ref, fp_ref,
                w1st_ref, b1s_ref, w2st_ref, b2s_ref,
                w1pt_ref, b1p_ref, w2pt_ref, b2p_ref,
                wst_ref, wpt_ref, woutt_ref, bout_ref, o_ref,
                ys_acc, yp0_acc, yp1_acc, yp2_acc):
    f32 = jnp.float32
    n = pl.program_id(0)

    xgt = jnp.transpose(xg_ref[...])
    gst = jnp.dot(wst_ref[...], xgt, preferred_element_type=f32)
    gpt = jnp.dot(wpt_ref[...], xgt, preferred_element_type=f32)
    mask = mask_ref[0]

    fsb = fs_ref[0]
    hs = _ssp(jnp.dot(w1st_ref[...], fsb, preferred_element_type=f32)
              + b1s_ref[...])
    hst = jnp.dot(w2st_ref[...], hs, preferred_element_type=f32) + b2s_ref[...]
    s_term = mask * (gst * hst)

    fpb = fp_ref[0]
    p_terms = []
    for k in range(3):
        hk = _ssp(jnp.dot(w1pt_ref[...], fpb[k * NG:(k + 1) * NG],
                          preferred_element_type=f32) + b1p_ref[...])
        hkt = jnp.dot(w2pt_ref[...], hk, preferred_element_type=f32) + b2p_ref[...]
        p_terms.append(mask * (gpt * hkt))

    @pl.when(n == 0)
    def _():
        ys_acc[...] = s_term
        yp0_acc[...] = p_terms[0]
        yp1_acc[...] = p_terms[1]
        yp2_acc[...] = p_terms[2]

    @pl.when(n > 0)
    def _():
        ys_acc[...] += s_term
        yp0_acc[...] += p_terms[0]
        yp1_acc[...] += p_terms[1]
        yp2_acc[...] += p_terms[2]

    @pl.when(n == Nnbh - 1)
    def _():
        y0, y1, y2 = yp0_acc[...], yp1_acc[...], yp2_acc[...]
        y = ys_acc[...] + y0 * y0 + y1 * y1 + y2 * y2
        out_t = jnp.dot(woutt_ref[...], y, preferred_element_type=f32) + bout_ref[...]
        o_ref[...] = jnp.transpose(out_t)


def _fused_call():
    def full(shape):
        return pl.BlockSpec(shape, lambda n: (0,) * len(shape))

    in_specs = [
        pl.BlockSpec((Na, NIN), lambda n: (n, 0)),
        pl.BlockSpec((1, 1, Na), lambda n: (n, 0, 0)),
        pl.BlockSpec((1, NG, Na), lambda n: (n, 0, 0)),
        pl.BlockSpec((1, 3 * NG, Na), lambda n: (n, 0, 0)),
        full((NF, NG)), full((NF, 1)), full((NF, NF)), full((NF, 1)),
        full((NF, NG)), full((NF, 1)), full((NF, NF)), full((NF, 1)),
        full((NF, NIN)), full((NF, NIN)), full((NOUT, NF)), full((NOUT, 1)),
    ]
    out_spec = pl.BlockSpec((Na, NOUT), lambda n: (0, 0))
    scratch = [pltpu.VMEM((NF, Na), jnp.float32) for _ in range(4)]
    return pl.pallas_call(
        _fused_body,
        grid=(Nnbh,),
        in_specs=in_specs,
        out_specs=out_spec,
        out_shape=jax.ShapeDtypeStruct((Na, NOUT), jnp.float32),
        scratch_shapes=scratch,
    )


def kernel(x, r_ij, neighbors, pairwise_mask, fsblock_ij, fpblock_ij,
           Wf1_s, bf1_s, Wf2_s, bf2_s, Wf1_p, bf1_p, Wf2_p, bf2_p,
           W_s, W_p, W_out, b_out):
    pad = (-(Na * Nnbh)) % (GATHER_WINDOW * SC_UNITS)
    zpad = jnp.zeros((pad,), jnp.int32)

    w1st = Wf1_s.T
    w2st = Wf2_s.T
    w1pt = Wf1_p.T
    w2pt = Wf2_p.T
    wst = W_s.T
    wpt = W_p.T
    woutt = W_out.T
    b1s = bf1_s.reshape(NF, 1)
    b2s = bf2_s.reshape(NF, 1)
    b1p = bf1_p.reshape(NF, 1)
    b2p = bf2_p.reshape(NF, 1)
    bout = b_out.reshape(NOUT, 1)

    call = _fused_call()
    outs = []
    for b in range(Nb):
        idx_b = jnp.concatenate(
            [jnp.swapaxes(neighbors[b], 0, 1).reshape(-1), zpad])
        xg_b = _sc_gather(x[b], idx_b)
        mask_t = jnp.swapaxes(pairwise_mask[b], 0, 1).reshape(Nnbh, 1, Na)
        fs_t = jnp.transpose(fsblock_ij[b], (1, 2, 3, 0)).reshape(Nnbh, NG, Na)
        fp_t = jnp.transpose(fpblock_ij[b], (1, 2, 3, 0)).reshape(Nnbh, 3 * NG, Na)
        y_b = call(xg_b, mask_t, fs_t, fp_t,
                   w1st, b1s, w2st, b2s,
                   w1pt, b1p, w2pt, b2p,
                   wst, wpt, woutt, bout)
        outs.append(y_b)
    return jnp.stack(outs)
